# Initial kernel scaffold; baseline (speedup 1.0000x reference)
import jax
import jax.numpy as jnp
from jax.experimental import pallas as pl


def kernel(unary, rgb, sp_map, spatial_ker_weights, bilateral_ker_weights, complex_rel_low_weights, complex_rel_high_weights, compatibility_matrix):
    raise NotImplementedError("write your pallas kernel here")


# speedup vs baseline: 12.3612x; 12.3612x over previous
import functools

import jax
import jax.numpy as jnp
from jax.experimental import pallas as pl

C = 21
L = 100
LP = 128
BN = 2048


def _softmax0(q):
    m = jnp.max(q, axis=0, keepdims=True)
    e = jnp.exp(q - m)
    return e / jnp.sum(e, axis=0, keepdims=True)


def _log_terms(s):
    logq = jnp.log(s + 1e-5)
    smax = jnp.max(s, axis=0, keepdims=True)
    maxc = jnp.max(s + 1e-5, axis=0, keepdims=True)
    bmax = (s == smax).astype(jnp.float32)
    a_sub = (s + 1e-5) + maxc - bmax * s
    loga = jnp.log(a_sub + 1e-5)
    return logq, loga, a_sub


def _onehot(seg_row):
    lidx = jax.lax.broadcasted_iota(jnp.int32, (LP, BN), 0)
    return (lidx == seg_row).astype(jnp.float32)


def _accumulate(oh, logq, loga, bsp_ref, bcont_ref):
    dn = (((1,), (1,)), ((), ()))
    bsp = jax.lax.dot_general(logq, oh, dn, preferred_element_type=jnp.float32)
    bcont = jax.lax.dot_general(loga, oh, dn, preferred_element_type=jnp.float32)

    @pl.when(pl.program_id(0) == 0)
    def _init():
        bsp_ref[...] = bsp
        bcont_ref[...] = bcont

    @pl.when(pl.program_id(0) != 0)
    def _acc():
        bsp_ref[...] += bsp
        bcont_ref[...] += bcont


def _update(q, s, oh, bsp, bcont, lwt, hw3, ws, wb, comp):
    dn = (((1,), (0,)), ((), ()))
    prod_sp = jax.lax.dot_general(bsp, oh, dn, preferred_element_type=jnp.float32)
    prod_io = jax.lax.dot_general(bcont, oh, dn, preferred_element_type=jnp.float32)
    prod_att = (L - 1.0) * prod_sp

    smax = jnp.max(s, axis=0, keepdims=True)
    maxc = jnp.max(s + 1e-5, axis=0, keepdims=True)
    bmax = (s == smax).astype(jnp.float32)
    a_sub = (s + 1e-5) + maxc - bmax * s
    q_mod = s + (s == 0).astype(jnp.float32)
    q_sum = a_sub + (a_sub == 0).astype(jnp.float32)
    log_qmod = jnp.log(q_mod + 1e-5)
    log_qsum = jnp.log(q_sum + 1e-5)

    ft_sp = jnp.exp(prod_sp - log_qmod)
    ft_io = jnp.exp(prod_io - log_qsum)
    ft_att = jnp.exp(prod_att - log_qmod)

    lw0 = lwt[:, 0:1]
    lw1 = lwt[:, 1:2]
    lw2 = lwt[:, 2:3]
    hw0 = hw3[0:1, 0:1]
    hw1 = hw3[0:1, 1:2]
    hw2 = hw3[0:1, 2:3]
    upd = (lw0 * ft_sp + hw0 * (1.0 - ft_sp)
           + lw1 * ft_io + hw1 * (1.0 - ft_io)
           + lw2 * ft_att + hw2 * (1.0 - ft_att))

    mw = jnp.dot(comp, ws + wb, preferred_element_type=jnp.float32)
    pw = jnp.dot(mw, s, preferred_element_type=jnp.float32)
    return pw + upd


def _kernel_a(q_ref, seg_ref, bsp_ref, bcont_ref):
    s = _softmax0(q_ref[...])
    logq, loga, _ = _log_terms(s)
    oh = _onehot(seg_ref[0])
    _accumulate(oh, logq, loga, bsp_ref, bcont_ref)


def _kernel_b(u_ref, seg_ref, bsp_ref, bcont_ref, lwt_ref, hw_ref, ws_ref,
              wb_ref, comp_ref, q1_ref, bsp2_ref, bcont2_ref):
    u = u_ref[...]
    s = _softmax0(u)
    oh = _onehot(seg_ref[0])
    delta = _update(u, s, oh, bsp_ref[...], bcont_ref[...], lwt_ref[...],
                    hw_ref[...], ws_ref[...], wb_ref[...], comp_ref[...])
    q1 = u - delta
    q1_ref[...] = q1
    s2 = _softmax0(q1)
    logq2, loga2, _ = _log_terms(s2)
    _accumulate(oh, logq2, loga2, bsp2_ref, bcont2_ref)


def _kernel_c(u_ref, q1_ref, seg_ref, bsp_ref, bcont_ref, lwt_ref, hw_ref,
              ws_ref, wb_ref, comp_ref, out_ref):
    q1 = q1_ref[...]
    s = _softmax0(q1)
    oh = _onehot(seg_ref[0])
    delta = _update(q1, s, oh, bsp_ref[...], bcont_ref[...], lwt_ref[...],
                    hw_ref[...], ws_ref[...], wb_ref[...], comp_ref[...])
    out_ref[...] = u_ref[...] - delta


def _crf_single(u_t, seg_row, lwt, hw3, ws, wb, comp, n, interpret=False):
    nb = n // BN
    grid = (nb,)
    qspec = pl.BlockSpec((C, BN), lambda i: (0, i))
    segspec = pl.BlockSpec((1, 1, BN), lambda i: (i, 0, 0))
    accspec = pl.BlockSpec((C, LP), lambda i: (0, 0))
    full = lambda a: pl.BlockSpec(a.shape, lambda i: (0,) * a.ndim)
    acc_shape = jax.ShapeDtypeStruct((C, LP), jnp.float32)
    q_shape = jax.ShapeDtypeStruct((C, n), jnp.float32)

    bsp1, bcont1 = pl.pallas_call(
        _kernel_a,
        grid=grid,
        in_specs=[qspec, segspec],
        out_specs=[accspec, accspec],
        out_shape=[acc_shape, acc_shape],
        interpret=interpret,
    )(u_t, seg_row)

    q1, bsp2, bcont2 = pl.pallas_call(
        _kernel_b,
        grid=grid,
        in_specs=[qspec, segspec, accspec, accspec, full(lwt), full(hw3),
                  full(ws), full(wb), full(comp)],
        out_specs=[qspec, accspec, accspec],
        out_shape=[q_shape, acc_shape, acc_shape],
        interpret=interpret,
    )(u_t, seg_row, bsp1, bcont1, lwt, hw3, ws, wb, comp)

    out = pl.pallas_call(
        _kernel_c,
        grid=grid,
        in_specs=[qspec, qspec, segspec, accspec, accspec, full(lwt),
                  full(hw3), full(ws), full(wb), full(comp)],
        out_specs=qspec,
        out_shape=q_shape,
        interpret=interpret,
    )(u_t, q1, seg_row, bsp2, bcont2, lwt, hw3, ws, wb, comp)
    return out


@functools.partial(jax.jit, static_argnames=("interpret",))
def _run(unary, sp_map, spatial_ker_weights, bilateral_ker_weights,
         complex_rel_low_weights, complex_rel_high_weights,
         compatibility_matrix, interpret=False):
    b, h, w, c = unary.shape
    n = h * w
    lwt = complex_rel_low_weights.T
    hw3 = complex_rel_high_weights.reshape(1, 3)
    outs = []
    for j in range(b):
        u_t = jnp.transpose(unary[j], (2, 0, 1)).reshape(c, n)
        seg_row = sp_map[j].reshape(n // BN, 1, BN).astype(jnp.int32)
        o = _crf_single(u_t, seg_row, lwt, hw3, spatial_ker_weights,
                        bilateral_ker_weights, compatibility_matrix, n,
                        interpret=interpret)
        outs.append(o.reshape(c, h, w))
    return jnp.stack(outs)


def kernel(unary, rgb, sp_map, spatial_ker_weights, bilateral_ker_weights,
           complex_rel_low_weights, complex_rel_high_weights,
           compatibility_matrix):
    return _run(unary, sp_map, spatial_ker_weights, bilateral_ker_weights,
                complex_rel_low_weights, complex_rel_high_weights,
                compatibility_matrix)


# speedup vs baseline: 18.3648x; 1.4857x over previous
import functools

import jax
import jax.numpy as jnp
from jax.experimental import pallas as pl

C = 21
L = 100
LP = 104
BN = 8192


def _softmax0(q):
    m = jnp.max(q, axis=0, keepdims=True)
    e = jnp.exp(q - m)
    z = jnp.sum(e, axis=0, keepdims=True)
    s = e / z
    bmax = (q == m)
    smax = 1.0 / z
    return s, bmax, smax


def _a_sub(s, bmax, smax):
    r = (smax + 1e-5) + 1e-5
    return jnp.where(bmax, jnp.broadcast_to(r, s.shape), s + r)


def _log_terms(s, bmax, smax):
    logq = jnp.log(s + 1e-5)
    loga = jnp.log(_a_sub(s, bmax, smax) + 1e-5)
    return logq, loga


def _onehot(seg_row):
    lidx = jax.lax.broadcasted_iota(jnp.int32, (LP, BN), 0)
    return (lidx == seg_row).astype(jnp.bfloat16)


def _accumulate(oh, logq, loga, bsp_ref, bcont_ref, bsp_acc, bcont_acc):
    dn = (((1,), (1,)), ((), ()))
    bsp = jax.lax.dot_general(logq.astype(jnp.bfloat16), oh, dn,
                              preferred_element_type=jnp.float32)
    bcont = jax.lax.dot_general(loga.astype(jnp.bfloat16), oh, dn,
                                preferred_element_type=jnp.float32)
    i = pl.program_id(0)

    @pl.when(i == 0)
    def _init():
        bsp_acc[...] = bsp
        bcont_acc[...] = bcont

    @pl.when(i != 0)
    def _acc():
        bsp_acc[...] += bsp
        bcont_acc[...] += bcont

    @pl.when(i == pl.num_programs(0) - 1)
    def _write():
        bsp_ref[...] = bsp_acc[...]
        bcont_ref[...] = bcont_acc[...]


def _update(s, bmax, smax, oh, bsp, bcont, lwt, hw3, ws, wb, comp):
    dn = (((1,), (0,)), ((), ()))
    prod_sp = jax.lax.dot_general(bsp.astype(jnp.bfloat16), oh, dn,
                                  preferred_element_type=jnp.float32)
    prod_io = jax.lax.dot_general(bcont.astype(jnp.bfloat16), oh, dn,
                                  preferred_element_type=jnp.float32)
    prod_att = (L - 1.0) * prod_sp

    a_sub = _a_sub(s, bmax, smax)
    q_mod = s + (s == 0).astype(jnp.float32)
    q_sum = a_sub + (a_sub == 0).astype(jnp.float32)
    log_qmod = jnp.log(q_mod + 1e-5)
    log_qsum = jnp.log(q_sum + 1e-5)

    ft_sp = jnp.exp(prod_sp - log_qmod)
    ft_io = jnp.exp(prod_io - log_qsum)
    ft_att = jnp.exp(prod_att - log_qmod)

    hw0 = hw3[0:1, 0:1]
    hw1 = hw3[0:1, 1:2]
    hw2 = hw3[0:1, 2:3]
    d0 = lwt[:, 0:1] - hw0
    d1 = lwt[:, 1:2] - hw1
    d2 = lwt[:, 2:3] - hw2
    upd = ((hw0 + hw1 + hw2) + d0 * ft_sp) + (d1 * ft_io + d2 * ft_att)

    mw = jnp.dot(comp, ws + wb, preferred_element_type=jnp.float32)
    pw = jnp.dot(mw.astype(jnp.bfloat16), s.astype(jnp.bfloat16),
                 preferred_element_type=jnp.float32)
    return pw + upd


def _kernel_a(q_ref, seg_ref, bsp_ref, bcont_ref, s_ref, smax_ref, bsp_acc,
              bcont_acc):
    s, bmax, smax = _softmax0(q_ref[...])
    s_ref[...] = s
    smax_ref[0] = smax
    logq, loga = _log_terms(s, bmax, smax)
    oh = _onehot(seg_ref[0])
    _accumulate(oh, logq, loga, bsp_ref, bcont_ref, bsp_acc, bcont_acc)


def _kernel_b(u_ref, s_ref, smax_ref, seg_ref, bsp_ref, bcont_ref, lwt_ref,
              hw_ref, ws_ref, wb_ref, comp_ref, s2_ref, smax2_ref, bsp2_ref,
              bcont2_ref, bsp2_acc, bcont2_acc):
    s = s_ref[...]
    smax = smax_ref[0]
    bmax = s == smax
    oh = _onehot(seg_ref[0])
    delta = _update(s, bmax, smax, oh, bsp_ref[...], bcont_ref[...],
                    lwt_ref[...], hw_ref[...], ws_ref[...], wb_ref[...],
                    comp_ref[...])
    q1 = u_ref[...] - delta
    s2, bmax2, smax2 = _softmax0(q1)
    s2_ref[...] = s2
    smax2_ref[0] = smax2
    logq2, loga2 = _log_terms(s2, bmax2, smax2)
    _accumulate(oh, logq2, loga2, bsp2_ref, bcont2_ref, bsp2_acc, bcont2_acc)


def _kernel_c(u_ref, s2_ref, smax2_ref, seg_ref, bsp_ref, bcont_ref, lwt_ref,
              hw_ref, ws_ref, wb_ref, comp_ref, out_ref):
    s = s2_ref[...]
    smax = smax2_ref[0]
    bmax = s == smax
    oh = _onehot(seg_ref[0])
    delta = _update(s, bmax, smax, oh, bsp_ref[...], bcont_ref[...],
                    lwt_ref[...], hw_ref[...], ws_ref[...], wb_ref[...],
                    comp_ref[...])
    out_ref[...] = u_ref[...] - delta


def _crf_single(u_t, seg_row, lwt, hw3, ws, wb, comp, n, interpret=False):
    nb = n // BN
    grid = (nb,)
    qspec = pl.BlockSpec((C, BN), lambda i: (0, i))
    segspec = pl.BlockSpec((1, 1, BN), lambda i: (i, 0, 0))
    accspec = pl.BlockSpec((C, LP), lambda i: (0, 0))
    full = lambda a: pl.BlockSpec(a.shape, lambda i: (0,) * a.ndim)
    acc_shape = jax.ShapeDtypeStruct((C, LP), jnp.float32)
    q_shape = jax.ShapeDtypeStruct((C, n), jnp.float32)
    row_shape = jax.ShapeDtypeStruct((nb, 1, BN), jnp.float32)

    from jax.experimental.pallas import tpu as pltpu
    acc_scratch = [pltpu.VMEM((C, LP), jnp.float32),
                   pltpu.VMEM((C, LP), jnp.float32)]

    bsp1, bcont1, s1, smax1 = pl.pallas_call(
        _kernel_a,
        grid=grid,
        in_specs=[qspec, segspec],
        out_specs=[accspec, accspec, qspec, segspec],
        out_shape=[acc_shape, acc_shape, q_shape, row_shape],
        scratch_shapes=acc_scratch,
        interpret=interpret,
    )(u_t, seg_row)

    s2, smax2, bsp2, bcont2 = pl.pallas_call(
        _kernel_b,
        grid=grid,
        in_specs=[qspec, qspec, segspec, segspec, accspec, accspec,
                  full(lwt), full(hw3), full(ws), full(wb), full(comp)],
        out_specs=[qspec, segspec, accspec, accspec],
        out_shape=[q_shape, row_shape, acc_shape, acc_shape],
        scratch_shapes=acc_scratch,
        interpret=interpret,
    )(u_t, s1, smax1, seg_row, bsp1, bcont1, lwt, hw3, ws, wb, comp)

    out = pl.pallas_call(
        _kernel_c,
        grid=grid,
        in_specs=[qspec, qspec, segspec, segspec, accspec, accspec,
                  full(lwt), full(hw3), full(ws), full(wb), full(comp)],
        out_specs=qspec,
        out_shape=q_shape,
        interpret=interpret,
    )(u_t, s2, smax2, seg_row, bsp2, bcont2, lwt, hw3, ws, wb, comp)
    return out


@functools.partial(jax.jit, static_argnames=("interpret",))
def _run(unary, sp_map, spatial_ker_weights, bilateral_ker_weights,
         complex_rel_low_weights, complex_rel_high_weights,
         compatibility_matrix, interpret=False):
    b, h, w, c = unary.shape
    n = h * w
    lwt = complex_rel_low_weights.T
    hw3 = complex_rel_high_weights.reshape(1, 3)
    outs = []
    for j in range(b):
        u_t = jnp.transpose(unary[j], (2, 0, 1)).reshape(c, n)
        seg_row = sp_map[j].reshape(n // BN, 1, BN).astype(jnp.int32)
        o = _crf_single(u_t, seg_row, lwt, hw3, spatial_ker_weights,
                        bilateral_ker_weights, compatibility_matrix, n,
                        interpret=interpret)
        outs.append(o.reshape(c, h, w))
    return jnp.stack(outs)


def kernel(unary, rgb, sp_map, spatial_ker_weights, bilateral_ker_weights,
           complex_rel_low_weights, complex_rel_high_weights,
           compatibility_matrix):
    return _run(unary, sp_map, spatial_ker_weights, bilateral_ker_weights,
                complex_rel_low_weights, complex_rel_high_weights,
                compatibility_matrix)


# speedup vs baseline: 18.4854x; 1.0066x over previous
import functools

import jax
import jax.numpy as jnp
from jax.experimental import pallas as pl

C = 21
L = 100
LP = 104
BN = 8192


def _softmax0(q):
    m = jnp.max(q, axis=0, keepdims=True)
    e = jnp.exp(q - m)
    z = jnp.sum(e, axis=0, keepdims=True)
    s = e / z
    smax = 1.0 / z
    return s, smax


def _a_sub(sb, bmax, smax_b):
    r = (smax_b.astype(jnp.float32) + 2e-5).astype(jnp.bfloat16)
    return jnp.where(bmax, jnp.broadcast_to(r, sb.shape), sb + r)


def _log_terms(sb, bmax, smax_b):
    logq = jnp.log(sb + jnp.bfloat16(1e-5))
    loga = jnp.log(_a_sub(sb, bmax, smax_b) + jnp.bfloat16(1e-5))
    return logq, loga


def _onehot(seg_row):
    lidx = jax.lax.broadcasted_iota(jnp.int32, (LP, BN), 0)
    return (lidx == seg_row).astype(jnp.bfloat16)


def _accumulate(oh, logq, loga, bsp_ref, bcont_ref, bsp_acc, bcont_acc):
    dn = (((1,), (1,)), ((), ()))
    bsp = jax.lax.dot_general(logq, oh, dn, preferred_element_type=jnp.float32)
    bcont = jax.lax.dot_general(loga, oh, dn, preferred_element_type=jnp.float32)
    i = pl.program_id(0)

    @pl.when(i == 0)
    def _init():
        bsp_acc[...] = bsp
        bcont_acc[...] = bcont

    @pl.when(i != 0)
    def _acc():
        bsp_acc[...] += bsp
        bcont_acc[...] += bcont

    @pl.when(i == pl.num_programs(0) - 1)
    def _write():
        bsp_ref[...] = bsp_acc[...]
        bcont_ref[...] = bcont_acc[...]


def _update(sb, bmax, smax, oh, bsp, bcont, lwt, hw3, ws, wb, comp):
    bf = jnp.bfloat16
    dn = (((1,), (0,)), ((), ()))
    prod_sp = jax.lax.dot_general(bsp.astype(bf), oh, dn,
                                  preferred_element_type=jnp.float32).astype(bf)
    prod_io = jax.lax.dot_general(bcont.astype(bf), oh, dn,
                                  preferred_element_type=jnp.float32).astype(bf)
    prod_att = bf(L - 1.0) * prod_sp

    a_sub = _a_sub(sb, bmax, smax)
    q_mod = sb + (sb == 0).astype(bf)
    q_sum = a_sub + (a_sub == 0).astype(bf)
    log_qmod = jnp.log(q_mod + bf(1e-5))
    log_qsum = jnp.log(q_sum + bf(1e-5))

    ft_sp = jnp.exp(prod_sp - log_qmod)
    ft_io = jnp.exp(prod_io - log_qsum)
    ft_att = jnp.exp(prod_att - log_qmod)

    hw0 = hw3[0:1, 0:1]
    hw1 = hw3[0:1, 1:2]
    hw2 = hw3[0:1, 2:3]
    d0 = (lwt[:, 0:1] - hw0).astype(bf)
    d1 = (lwt[:, 1:2] - hw1).astype(bf)
    d2 = (lwt[:, 2:3] - hw2).astype(bf)
    ftsum = (d0 * ft_sp + d1 * ft_io) + d2 * ft_att
    upd = (hw0 + hw1 + hw2) + ftsum.astype(jnp.float32)

    mw = jnp.dot(comp, ws + wb, preferred_element_type=jnp.float32)
    pw = jnp.dot(mw.astype(bf), sb, preferred_element_type=jnp.float32)
    return pw + upd


def _kernel_a(q_ref, seg_ref, bsp_ref, bcont_ref, s_ref, smax_ref, bsp_acc,
              bcont_acc):
    s, smax = _softmax0(q_ref[...])
    sb = s.astype(jnp.bfloat16)
    smax_b = smax.astype(jnp.bfloat16)
    s_ref[...] = sb
    smax_ref[0] = smax_b
    logq, loga = _log_terms(sb, sb == smax_b, smax_b)
    oh = _onehot(seg_ref[0])
    _accumulate(oh, logq, loga, bsp_ref, bcont_ref, bsp_acc, bcont_acc)


def _kernel_b(u_ref, s_ref, smax_ref, seg_ref, bsp_ref, bcont_ref, lwt_ref,
              hw_ref, ws_ref, wb_ref, comp_ref, s2_ref, smax2_ref, bsp2_ref,
              bcont2_ref, bsp2_acc, bcont2_acc):
    smax_b = smax_ref[0]
    sb = s_ref[...]
    oh = _onehot(seg_ref[0])
    delta = _update(sb, sb == smax_b, smax_b, oh, bsp_ref[...],
                    bcont_ref[...], lwt_ref[...], hw_ref[...], ws_ref[...],
                    wb_ref[...], comp_ref[...])
    q1 = u_ref[...] - delta
    s2, smax2 = _softmax0(q1)
    sb2 = s2.astype(jnp.bfloat16)
    smax2_b = smax2.astype(jnp.bfloat16)
    s2_ref[...] = sb2
    smax2_ref[0] = smax2_b
    logq2, loga2 = _log_terms(sb2, sb2 == smax2_b, smax2_b)
    _accumulate(oh, logq2, loga2, bsp2_ref, bcont2_ref, bsp2_acc, bcont2_acc)


def _kernel_c(u_ref, s2_ref, smax2_ref, seg_ref, bsp_ref, bcont_ref, lwt_ref,
              hw_ref, ws_ref, wb_ref, comp_ref, out_ref):
    smax_b = smax2_ref[0]
    sb = s2_ref[...]
    oh = _onehot(seg_ref[0])
    delta = _update(sb, sb == smax_b, smax_b, oh, bsp_ref[...],
                    bcont_ref[...], lwt_ref[...], hw_ref[...], ws_ref[...],
                    wb_ref[...], comp_ref[...])
    out_ref[...] = u_ref[...] - delta


def _crf_single(u_t, seg_row, lwt, hw3, ws, wb, comp, n, interpret=False):
    nb = n // BN
    grid = (nb,)
    qspec = pl.BlockSpec((C, BN), lambda i: (0, i))
    segspec = pl.BlockSpec((1, 1, BN), lambda i: (i, 0, 0))
    accspec = pl.BlockSpec((C, LP), lambda i: (0, 0))
    full = lambda a: pl.BlockSpec(a.shape, lambda i: (0,) * a.ndim)
    acc_shape = jax.ShapeDtypeStruct((C, LP), jnp.float32)
    q_shape = jax.ShapeDtypeStruct((C, n), jnp.float32)
    s_shape = jax.ShapeDtypeStruct((C, n), jnp.bfloat16)
    row_shape = jax.ShapeDtypeStruct((nb, 1, BN), jnp.bfloat16)

    from jax.experimental.pallas import tpu as pltpu
    acc_scratch = [pltpu.VMEM((C, LP), jnp.float32),
                   pltpu.VMEM((C, LP), jnp.float32)]

    bsp1, bcont1, s1, smax1 = pl.pallas_call(
        _kernel_a,
        grid=grid,
        in_specs=[qspec, segspec],
        out_specs=[accspec, accspec, qspec, segspec],
        out_shape=[acc_shape, acc_shape, s_shape, row_shape],
        scratch_shapes=acc_scratch,
        interpret=interpret,
    )(u_t, seg_row)

    s2, smax2, bsp2, bcont2 = pl.pallas_call(
        _kernel_b,
        grid=grid,
        in_specs=[qspec, qspec, segspec, segspec, accspec, accspec,
                  full(lwt), full(hw3), full(ws), full(wb), full(comp)],
        out_specs=[qspec, segspec, accspec, accspec],
        out_shape=[s_shape, row_shape, acc_shape, acc_shape],
        scratch_shapes=acc_scratch,
        interpret=interpret,
    )(u_t, s1, smax1, seg_row, bsp1, bcont1, lwt, hw3, ws, wb, comp)

    out = pl.pallas_call(
        _kernel_c,
        grid=grid,
        in_specs=[qspec, qspec, segspec, segspec, accspec, accspec,
                  full(lwt), full(hw3), full(ws), full(wb), full(comp)],
        out_specs=qspec,
        out_shape=q_shape,
        interpret=interpret,
    )(u_t, s2, smax2, seg_row, bsp2, bcont2, lwt, hw3, ws, wb, comp)
    return out


@functools.partial(jax.jit, static_argnames=("interpret",))
def _run(unary, sp_map, spatial_ker_weights, bilateral_ker_weights,
         complex_rel_low_weights, complex_rel_high_weights,
         compatibility_matrix, interpret=False):
    b, h, w, c = unary.shape
    n = h * w
    lwt = complex_rel_low_weights.T
    hw3 = complex_rel_high_weights.reshape(1, 3)
    outs = []
    for j in range(b):
        u_t = jnp.transpose(unary[j], (2, 0, 1)).reshape(c, n)
        seg_row = sp_map[j].reshape(n // BN, 1, BN).astype(jnp.int32)
        o = _crf_single(u_t, seg_row, lwt, hw3, spatial_ker_weights,
                        bilateral_ker_weights, compatibility_matrix, n,
                        interpret=interpret)
        outs.append(o.reshape(c, h, w))
    return jnp.stack(outs)


def kernel(unary, rgb, sp_map, spatial_ker_weights, bilateral_ker_weights,
           complex_rel_low_weights, complex_rel_high_weights,
           compatibility_matrix):
    return _run(unary, sp_map, spatial_ker_weights, bilateral_ker_weights,
                complex_rel_low_weights, complex_rel_high_weights,
                compatibility_matrix)


# speedup vs baseline: 20.7387x; 1.1219x over previous
import functools

import jax
import jax.numpy as jnp
from jax.experimental import pallas as pl

C = 21
CP = 24
L = 100
LP = 104
BN = 8192


def _softmax0(q):
    m = jnp.max(q, axis=0, keepdims=True)
    e = jnp.exp(q - m)
    z = jnp.sum(e, axis=0, keepdims=True)
    s = e / z
    smax = 1.0 / z
    return s, smax


def _a_sub(sb, bmax, smax_b):
    r = (smax_b.astype(jnp.float32) + 2e-5).astype(jnp.bfloat16)
    return jnp.where(bmax, jnp.broadcast_to(r, sb.shape), sb + r)


def _log_terms(sb, bmax, smax_b):
    pad = jnp.ones((CP - C, sb.shape[1]), jnp.bfloat16)
    stack = jnp.concatenate(
        [sb + jnp.bfloat16(1e-5), pad,
         _a_sub(sb, bmax, smax_b) + jnp.bfloat16(1e-5), pad], axis=0)
    return jnp.log(stack)


def _onehot(seg_row):
    lidx = jax.lax.broadcasted_iota(jnp.int32, (LP, BN), 0)
    return (lidx == seg_row).astype(jnp.bfloat16)


def _accumulate(oh, lq_all, bc_ref, bc_acc):
    dn = (((1,), (1,)), ((), ()))
    bc = jax.lax.dot_general(lq_all, oh, dn, preferred_element_type=jnp.float32)
    i = pl.program_id(0)

    @pl.when(i == 0)
    def _init():
        bc_acc[...] = bc

    @pl.when(i != 0)
    def _acc():
        bc_acc[...] += bc

    @pl.when(i == pl.num_programs(0) - 1)
    def _write():
        bc_ref[...] = bc_acc[...]


def _update(sb, bmax, smax, oh, bc, lwt, hw3, ws, wb, comp):
    bf = jnp.bfloat16
    dn = (((1,), (0,)), ((), ()))
    prod_all = jax.lax.dot_general(
        bc.astype(bf), oh, dn,
        preferred_element_type=jnp.float32).astype(bf)
    prod_sp = prod_all[0:C]
    prod_io = prod_all[CP:CP + C]
    prod_att = bf(L - 1.0) * prod_sp

    a_sub = _a_sub(sb, bmax, smax)
    q_mod = sb + (sb == 0).astype(bf)
    log_qmod = jnp.log(q_mod + bf(1e-5))
    log_qsum = jnp.log(a_sub + bf(1e-5))

    ft_sp = jnp.exp(prod_sp - log_qmod)
    ft_io = jnp.exp(prod_io - log_qsum)
    ft_att = jnp.exp(prod_att - log_qmod)

    hw0 = hw3[0:1, 0:1]
    hw1 = hw3[0:1, 1:2]
    hw2 = hw3[0:1, 2:3]
    d0 = (lwt[:, 0:1] - hw0).astype(bf)
    d1 = (lwt[:, 1:2] - hw1).astype(bf)
    d2 = (lwt[:, 2:3] - hw2).astype(bf)
    ftsum = (d0 * ft_sp + d1 * ft_io) + d2 * ft_att
    upd = (hw0 + hw1 + hw2) + ftsum.astype(jnp.float32)

    mw = jnp.dot(comp, ws + wb, preferred_element_type=jnp.float32)
    pw = jnp.dot(mw.astype(bf), sb, preferred_element_type=jnp.float32)
    return pw + upd


def _kernel_a(q_ref, seg_ref, bc_ref, s_ref, smax_ref, bc_acc):
    s, smax = _softmax0(q_ref[...])
    sb = s.astype(jnp.bfloat16)
    smax_b = smax.astype(jnp.bfloat16)
    s_ref[...] = sb
    smax_ref[0] = smax_b
    lq_all = _log_terms(sb, sb == smax_b, smax_b)
    oh = _onehot(seg_ref[0])
    _accumulate(oh, lq_all, bc_ref, bc_acc)


def _kernel_b(u_ref, s_ref, smax_ref, seg_ref, bc_ref, lwt_ref,
              hw_ref, ws_ref, wb_ref, comp_ref, s2_ref, smax2_ref, bc2_ref,
              bc2_acc):
    smax_b = smax_ref[0]
    sb = s_ref[...]
    oh = _onehot(seg_ref[0])
    delta = _update(sb, sb == smax_b, smax_b, oh, bc_ref[...], lwt_ref[...],
                    hw_ref[...], ws_ref[...], wb_ref[...], comp_ref[...])
    q1 = u_ref[...] - delta
    s2, smax2 = _softmax0(q1)
    sb2 = s2.astype(jnp.bfloat16)
    smax2_b = smax2.astype(jnp.bfloat16)
    s2_ref[...] = sb2
    smax2_ref[0] = smax2_b
    lq_all2 = _log_terms(sb2, sb2 == smax2_b, smax2_b)
    _accumulate(oh, lq_all2, bc2_ref, bc2_acc)


def _kernel_c(u_ref, s2_ref, smax2_ref, seg_ref, bc_ref, lwt_ref,
              hw_ref, ws_ref, wb_ref, comp_ref, out_ref):
    smax_b = smax2_ref[0]
    sb = s2_ref[...]
    oh = _onehot(seg_ref[0])
    delta = _update(sb, sb == smax_b, smax_b, oh, bc_ref[...], lwt_ref[...],
                    hw_ref[...], ws_ref[...], wb_ref[...], comp_ref[...])
    out_ref[...] = u_ref[...] - delta


def _crf_single(u_t, seg_row, lwt, hw3, ws, wb, comp, n, interpret=False):
    nb = n // BN
    grid = (nb,)
    qspec = pl.BlockSpec((C, BN), lambda i: (0, i))
    segspec = pl.BlockSpec((1, 1, BN), lambda i: (i, 0, 0))
    accspec = pl.BlockSpec((2 * CP, LP), lambda i: (0, 0))
    full = lambda a: pl.BlockSpec(a.shape, lambda i: (0,) * a.ndim)
    acc_shape = jax.ShapeDtypeStruct((2 * CP, LP), jnp.float32)
    q_shape = jax.ShapeDtypeStruct((C, n), jnp.float32)
    s_shape = jax.ShapeDtypeStruct((C, n), jnp.bfloat16)
    row_shape = jax.ShapeDtypeStruct((nb, 1, BN), jnp.bfloat16)

    from jax.experimental.pallas import tpu as pltpu
    acc_scratch = [pltpu.VMEM((2 * CP, LP), jnp.float32)]

    bc1, s1, smax1 = pl.pallas_call(
        _kernel_a,
        grid=grid,
        in_specs=[qspec, segspec],
        out_specs=[accspec, qspec, segspec],
        out_shape=[acc_shape, s_shape, row_shape],
        scratch_shapes=acc_scratch,
        interpret=interpret,
    )(u_t, seg_row)

    s2, smax2, bc2 = pl.pallas_call(
        _kernel_b,
        grid=grid,
        in_specs=[qspec, qspec, segspec, segspec, accspec,
                  full(lwt), full(hw3), full(ws), full(wb), full(comp)],
        out_specs=[qspec, segspec, accspec],
        out_shape=[s_shape, row_shape, acc_shape],
        scratch_shapes=acc_scratch,
        interpret=interpret,
    )(u_t, s1, smax1, seg_row, bc1, lwt, hw3, ws, wb, comp)

    out = pl.pallas_call(
        _kernel_c,
        grid=grid,
        in_specs=[qspec, qspec, segspec, segspec, accspec,
                  full(lwt), full(hw3), full(ws), full(wb), full(comp)],
        out_specs=qspec,
        out_shape=q_shape,
        interpret=interpret,
    )(u_t, s2, smax2, seg_row, bc2, lwt, hw3, ws, wb, comp)
    return out


@functools.partial(jax.jit, static_argnames=("interpret",))
def _run(unary, sp_map, spatial_ker_weights, bilateral_ker_weights,
         complex_rel_low_weights, complex_rel_high_weights,
         compatibility_matrix, interpret=False):
    b, h, w, c = unary.shape
    n = h * w
    lwt = complex_rel_low_weights.T
    hw3 = complex_rel_high_weights.reshape(1, 3)
    outs = []
    for j in range(b):
        u_t = jnp.transpose(unary[j], (2, 0, 1)).reshape(c, n)
        seg_row = sp_map[j].reshape(n // BN, 1, BN).astype(jnp.int32)
        o = _crf_single(u_t, seg_row, lwt, hw3, spatial_ker_weights,
                        bilateral_ker_weights, compatibility_matrix, n,
                        interpret=interpret)
        outs.append(o.reshape(c, h, w))
    return jnp.stack(outs)


def kernel(unary, rgb, sp_map, spatial_ker_weights, bilateral_ker_weights,
           complex_rel_low_weights, complex_rel_high_weights,
           compatibility_matrix):
    return _run(unary, sp_map, spatial_ker_weights, bilateral_ker_weights,
                complex_rel_low_weights, complex_rel_high_weights,
                compatibility_matrix)


# speedup vs baseline: 21.6871x; 1.0457x over previous
import functools

import jax
import jax.numpy as jnp
from jax.experimental import pallas as pl

C = 21
CP = 24
L = 100
LP = 104
BN = 16384


def _softmax0(q):
    m = jnp.max(q, axis=0, keepdims=True)
    e = jnp.exp(q - m)
    z = jnp.sum(e, axis=0, keepdims=True)
    s = e / z
    smax = 1.0 / z
    return s, smax


def _a_sub(sb, bmax, smax_b):
    r = (smax_b.astype(jnp.float32) + 2e-5).astype(jnp.bfloat16)
    return jnp.where(bmax, jnp.broadcast_to(r, sb.shape), sb + r)


def _log_terms(sb, bmax, smax_b):
    pad = jnp.ones((CP - C, sb.shape[1]), jnp.bfloat16)
    stack = jnp.concatenate(
        [sb + jnp.bfloat16(1e-5), pad,
         _a_sub(sb, bmax, smax_b) + jnp.bfloat16(1e-5), pad], axis=0)
    return jnp.log(stack)


def _onehot(seg_row):
    lidx = jax.lax.broadcasted_iota(jnp.int32, (LP, BN), 0)
    return (lidx == seg_row).astype(jnp.bfloat16)


def _accumulate(oh, lq_all, bc_ref, bc_acc):
    dn = (((1,), (1,)), ((), ()))
    bc = jax.lax.dot_general(lq_all, oh, dn, preferred_element_type=jnp.float32)
    i = pl.program_id(0)

    @pl.when(i == 0)
    def _init():
        bc_acc[...] = bc

    @pl.when(i != 0)
    def _acc():
        bc_acc[...] += bc

    @pl.when(i == pl.num_programs(0) - 1)
    def _write():
        bc_ref[...] = bc_acc[...]


def _update(sb, bmax, smax, oh, bc, lwt, hw3, ws, wb, comp):
    bf = jnp.bfloat16
    dn = (((1,), (0,)), ((), ()))
    prod_all = jax.lax.dot_general(
        bc.astype(bf), oh, dn,
        preferred_element_type=jnp.float32).astype(bf)
    prod_sp = prod_all[0:C]
    prod_io = prod_all[CP:CP + C]
    prod_att = bf(L - 1.0) * prod_sp

    a_sub = _a_sub(sb, bmax, smax)
    q_mod = sb + (sb == 0).astype(bf)
    log_qmod = jnp.log(q_mod + bf(1e-5))
    log_qsum = jnp.log(a_sub + bf(1e-5))

    ft_sp = jnp.exp(prod_sp - log_qmod)
    ft_io = jnp.exp(prod_io - log_qsum)
    ft_att = jnp.exp(prod_att - log_qmod)

    hw0 = hw3[0:1, 0:1]
    hw1 = hw3[0:1, 1:2]
    hw2 = hw3[0:1, 2:3]
    d0 = (lwt[:, 0:1] - hw0).astype(bf)
    d1 = (lwt[:, 1:2] - hw1).astype(bf)
    d2 = (lwt[:, 2:3] - hw2).astype(bf)
    ftsum = (d0 * ft_sp + d1 * ft_io) + d2 * ft_att
    upd = (hw0 + hw1 + hw2) + ftsum.astype(jnp.float32)

    mw = jnp.dot(comp, ws + wb, preferred_element_type=jnp.float32)
    pw = jnp.dot(mw.astype(bf), sb, preferred_element_type=jnp.float32)
    return pw + upd


def _kernel_a(q_ref, seg_ref, bc_ref, s_ref, smax_ref, bc_acc):
    s, smax = _softmax0(q_ref[...])
    sb = s.astype(jnp.bfloat16)
    smax_b = smax.astype(jnp.bfloat16)
    s_ref[...] = sb
    smax_ref[0] = smax_b
    lq_all = _log_terms(sb, sb == smax_b, smax_b)
    oh = _onehot(seg_ref[0])
    _accumulate(oh, lq_all, bc_ref, bc_acc)


def _kernel_b(u_ref, s_ref, smax_ref, seg_ref, bc_ref, lwt_ref,
              hw_ref, ws_ref, wb_ref, comp_ref, s2_ref, smax2_ref, bc2_ref,
              bc2_acc):
    smax_b = smax_ref[0]
    sb = s_ref[...]
    oh = _onehot(seg_ref[0])
    delta = _update(sb, sb == smax_b, smax_b, oh, bc_ref[...], lwt_ref[...],
                    hw_ref[...], ws_ref[...], wb_ref[...], comp_ref[...])
    q1 = u_ref[...] - delta
    s2, smax2 = _softmax0(q1)
    sb2 = s2.astype(jnp.bfloat16)
    smax2_b = smax2.astype(jnp.bfloat16)
    s2_ref[...] = sb2
    smax2_ref[0] = smax2_b
    lq_all2 = _log_terms(sb2, sb2 == smax2_b, smax2_b)
    _accumulate(oh, lq_all2, bc2_ref, bc2_acc)


def _kernel_c(u_ref, s2_ref, smax2_ref, seg_ref, bc_ref, lwt_ref,
              hw_ref, ws_ref, wb_ref, comp_ref, out_ref):
    smax_b = smax2_ref[0]
    sb = s2_ref[...]
    oh = _onehot(seg_ref[0])
    delta = _update(sb, sb == smax_b, smax_b, oh, bc_ref[...], lwt_ref[...],
                    hw_ref[...], ws_ref[...], wb_ref[...], comp_ref[...])
    out_ref[...] = u_ref[...] - delta


def _crf_single(u_t, seg_row, lwt, hw3, ws, wb, comp, n, interpret=False):
    nb = n // BN
    grid = (nb,)
    qspec = pl.BlockSpec((C, BN), lambda i: (0, i))
    segspec = pl.BlockSpec((1, 1, BN), lambda i: (i, 0, 0))
    accspec = pl.BlockSpec((2 * CP, LP), lambda i: (0, 0))
    full = lambda a: pl.BlockSpec(a.shape, lambda i: (0,) * a.ndim)
    acc_shape = jax.ShapeDtypeStruct((2 * CP, LP), jnp.float32)
    q_shape = jax.ShapeDtypeStruct((C, n), jnp.float32)
    s_shape = jax.ShapeDtypeStruct((C, n), jnp.bfloat16)
    row_shape = jax.ShapeDtypeStruct((nb, 1, BN), jnp.bfloat16)

    from jax.experimental.pallas import tpu as pltpu
    acc_scratch = [pltpu.VMEM((2 * CP, LP), jnp.float32)]

    bc1, s1, smax1 = pl.pallas_call(
        _kernel_a,
        grid=grid,
        in_specs=[qspec, segspec],
        out_specs=[accspec, qspec, segspec],
        out_shape=[acc_shape, s_shape, row_shape],
        scratch_shapes=acc_scratch,
        interpret=interpret,
    )(u_t, seg_row)

    s2, smax2, bc2 = pl.pallas_call(
        _kernel_b,
        grid=grid,
        in_specs=[qspec, qspec, segspec, segspec, accspec,
                  full(lwt), full(hw3), full(ws), full(wb), full(comp)],
        out_specs=[qspec, segspec, accspec],
        out_shape=[s_shape, row_shape, acc_shape],
        scratch_shapes=acc_scratch,
        interpret=interpret,
    )(u_t, s1, smax1, seg_row, bc1, lwt, hw3, ws, wb, comp)

    out = pl.pallas_call(
        _kernel_c,
        grid=grid,
        in_specs=[qspec, qspec, segspec, segspec, accspec,
                  full(lwt), full(hw3), full(ws), full(wb), full(comp)],
        out_specs=qspec,
        out_shape=q_shape,
        interpret=interpret,
    )(u_t, s2, smax2, seg_row, bc2, lwt, hw3, ws, wb, comp)
    return out


@functools.partial(jax.jit, static_argnames=("interpret",))
def _run(unary, sp_map, spatial_ker_weights, bilateral_ker_weights,
         complex_rel_low_weights, complex_rel_high_weights,
         compatibility_matrix, interpret=False):
    b, h, w, c = unary.shape
    n = h * w
    lwt = complex_rel_low_weights.T
    hw3 = complex_rel_high_weights.reshape(1, 3)
    outs = []
    for j in range(b):
        u_t = jnp.transpose(unary[j], (2, 0, 1)).reshape(c, n)
        seg_row = sp_map[j].reshape(n // BN, 1, BN).astype(jnp.int32)
        o = _crf_single(u_t, seg_row, lwt, hw3, spatial_ker_weights,
                        bilateral_ker_weights, compatibility_matrix, n,
                        interpret=interpret)
        outs.append(o.reshape(c, h, w))
    return jnp.stack(outs)


def kernel(unary, rgb, sp_map, spatial_ker_weights, bilateral_ker_weights,
           complex_rel_low_weights, complex_rel_high_weights,
           compatibility_matrix):
    return _run(unary, sp_map, spatial_ker_weights, bilateral_ker_weights,
                complex_rel_low_weights, complex_rel_high_weights,
                compatibility_matrix)


# speedup vs baseline: 21.9679x; 1.0129x over previous
import functools

import jax
import jax.numpy as jnp
from jax.experimental import pallas as pl
from jax.experimental.pallas import tpu as pltpu

C = 21
CP = 24
L = 100
LP = 104
BN = 16384


def _softmax0(q):
    m = jnp.max(q, axis=0, keepdims=True)
    e = jnp.exp(q - m)
    z = jnp.sum(e, axis=0, keepdims=True)
    s = e / z
    smax = 1.0 / z
    return s, smax


def _a_sub(sb, bmax, smax_b):
    r = (smax_b.astype(jnp.float32) + 2e-5).astype(jnp.bfloat16)
    return jnp.where(bmax, jnp.broadcast_to(r, sb.shape), sb + r)


def _log_terms(sb, bmax, smax_b):
    pad = jnp.ones((CP - C, sb.shape[1]), jnp.bfloat16)
    stack = jnp.concatenate(
        [sb + jnp.bfloat16(1e-5), pad,
         _a_sub(sb, bmax, smax_b) + jnp.bfloat16(1e-5), pad], axis=0)
    return jnp.log(stack)


def _onehot(seg_row):
    lidx = jax.lax.broadcasted_iota(jnp.int32, (LP, BN), 0)
    return (lidx == seg_row).astype(jnp.bfloat16)


def _accumulate(oh, lq_all, bc_acc, i):
    dn = (((1,), (1,)), ((), ()))
    bc = jax.lax.dot_general(lq_all, oh, dn, preferred_element_type=jnp.float32)

    @pl.when(i == 0)
    def _init():
        bc_acc[...] = bc

    @pl.when(i != 0)
    def _acc():
        bc_acc[...] += bc


def _update(sb, bmax, smax, oh, bc, lwt, hw3, ws, wb, comp):
    bf = jnp.bfloat16
    dn = (((1,), (0,)), ((), ()))
    prod_all = jax.lax.dot_general(
        bc.astype(bf), oh, dn,
        preferred_element_type=jnp.float32).astype(bf)
    prod_sp = prod_all[0:C]
    prod_io = prod_all[CP:CP + C]
    prod_att = bf(L - 1.0) * prod_sp

    a_sub = _a_sub(sb, bmax, smax)
    q_mod = sb + (sb == 0).astype(bf)
    log_qmod = jnp.log(q_mod + bf(1e-5))
    log_qsum = jnp.log(a_sub + bf(1e-5))

    ft_sp = jnp.exp(prod_sp - log_qmod)
    ft_io = jnp.exp(prod_io - log_qsum)
    ft_att = jnp.exp(prod_att - log_qmod)

    hw0 = hw3[0:1, 0:1]
    hw1 = hw3[0:1, 1:2]
    hw2 = hw3[0:1, 2:3]
    d0 = (lwt[:, 0:1] - hw0).astype(bf)
    d1 = (lwt[:, 1:2] - hw1).astype(bf)
    d2 = (lwt[:, 2:3] - hw2).astype(bf)
    ftsum = (d0 * ft_sp + d1 * ft_io) + d2 * ft_att
    upd = (hw0 + hw1 + hw2) + ftsum.astype(jnp.float32)

    mw = jnp.dot(comp, ws + wb, preferred_element_type=jnp.float32)
    pw = jnp.dot(mw.astype(bf), sb, preferred_element_type=jnp.float32)
    return pw + upd


def _kernel_all(u_ref, seg_ref, lwt_ref, hw_ref, ws_ref, wb_ref, comp_ref,
                out_ref, s1_all, smax1_all, s2_all, smax2_all, bc1_acc,
                bc2_acc):
    ph = pl.program_id(0)
    i = pl.program_id(1)
    sl = pl.ds(i * BN, BN)
    oh = _onehot(seg_ref[0])

    @pl.when(ph == 0)
    def _p0():
        s, smax = _softmax0(u_ref[...])
        sb = s.astype(jnp.bfloat16)
        smax_b = smax.astype(jnp.bfloat16)
        s1_all[:, sl] = sb
        smax1_all[:, sl] = smax_b
        lq_all = _log_terms(sb, sb == smax_b, smax_b)
        _accumulate(oh, lq_all, bc1_acc, i)

    @pl.when(ph == 1)
    def _p1():
        sb = s1_all[:, sl]
        smax_b = smax1_all[:, sl]
        delta = _update(sb, sb == smax_b, smax_b, oh, bc1_acc[...],
                        lwt_ref[...], hw_ref[...], ws_ref[...], wb_ref[...],
                        comp_ref[...])
        q1 = u_ref[...] - delta
        s2, smax2 = _softmax0(q1)
        sb2 = s2.astype(jnp.bfloat16)
        smax2_b = smax2.astype(jnp.bfloat16)
        s2_all[:, sl] = sb2
        smax2_all[:, sl] = smax2_b
        lq_all2 = _log_terms(sb2, sb2 == smax2_b, smax2_b)
        _accumulate(oh, lq_all2, bc2_acc, i)

    @pl.when(ph == 2)
    def _p2():
        sb = s2_all[:, sl]
        smax_b = smax2_all[:, sl]
        delta = _update(sb, sb == smax_b, smax_b, oh, bc2_acc[...],
                        lwt_ref[...], hw_ref[...], ws_ref[...], wb_ref[...],
                        comp_ref[...])
        out_ref[...] = u_ref[...] - delta


def _crf_single(u_t, seg_row, lwt, hw3, ws, wb, comp, n, interpret=False):
    nb = n // BN
    qspec = pl.BlockSpec((C, BN), lambda ph, i: (0, i))
    segspec = pl.BlockSpec((1, 1, BN), lambda ph, i: (i, 0, 0))
    full = lambda a: pl.BlockSpec(a.shape, lambda ph, i: (0,) * a.ndim)

    return pl.pallas_call(
        _kernel_all,
        grid=(3, nb),
        in_specs=[qspec, segspec, full(lwt), full(hw3), full(ws), full(wb),
                  full(comp)],
        out_specs=qspec,
        out_shape=jax.ShapeDtypeStruct((C, n), jnp.float32),
        scratch_shapes=[
            pltpu.VMEM((C, n), jnp.bfloat16),
            pltpu.VMEM((1, n), jnp.bfloat16),
            pltpu.VMEM((C, n), jnp.bfloat16),
            pltpu.VMEM((1, n), jnp.bfloat16),
            pltpu.VMEM((2 * CP, LP), jnp.float32),
            pltpu.VMEM((2 * CP, LP), jnp.float32),
        ],
        interpret=interpret,
    )(u_t, seg_row, lwt, hw3, ws, wb, comp)


@functools.partial(jax.jit, static_argnames=("interpret",))
def _run(unary, sp_map, spatial_ker_weights, bilateral_ker_weights,
         complex_rel_low_weights, complex_rel_high_weights,
         compatibility_matrix, interpret=False):
    b, h, w, c = unary.shape
    n = h * w
    lwt = complex_rel_low_weights.T
    hw3 = complex_rel_high_weights.reshape(1, 3)
    outs = []
    for j in range(b):
        u_t = jnp.transpose(unary[j], (2, 0, 1)).reshape(c, n)
        seg_row = sp_map[j].reshape(n // BN, 1, BN).astype(jnp.int32)
        o = _crf_single(u_t, seg_row, lwt, hw3, spatial_ker_weights,
                        bilateral_ker_weights, compatibility_matrix, n,
                        interpret=interpret)
        outs.append(o.reshape(c, h, w))
    return jnp.stack(outs)


def kernel(unary, rgb, sp_map, spatial_ker_weights, bilateral_ker_weights,
           complex_rel_low_weights, complex_rel_high_weights,
           compatibility_matrix):
    return _run(unary, sp_map, spatial_ker_weights, bilateral_ker_weights,
                complex_rel_low_weights, complex_rel_high_weights,
                compatibility_matrix)


# speedup vs baseline: 22.0136x; 1.0021x over previous
import functools

import jax
import jax.numpy as jnp
from jax.experimental import pallas as pl
from jax.experimental.pallas import tpu as pltpu

C = 21
CP = 24
L = 100
LP = 104
BN = 16384


def _softmax0(q):
    m = jnp.max(q, axis=0, keepdims=True)
    e = jnp.exp(q - m)
    z = jnp.sum(e, axis=0, keepdims=True)
    s = e / z
    smax = 1.0 / z
    return s, smax


def _a_sub(sb, bmax, smax_b):
    r = (smax_b.astype(jnp.float32) + 2e-5).astype(jnp.bfloat16)
    return jnp.where(bmax, jnp.broadcast_to(r, sb.shape), sb + r)


def _log_terms(sb, bmax, smax_b):
    pad = jnp.ones((CP - C, sb.shape[1]), jnp.bfloat16)
    stack = jnp.concatenate(
        [sb + jnp.bfloat16(1e-5), pad,
         _a_sub(sb, bmax, smax_b) + jnp.bfloat16(1e-5), pad], axis=0)
    return jnp.log(stack)


def _onehot(seg_row):
    lidx = jax.lax.broadcasted_iota(jnp.int32, (LP, BN), 0)
    return (lidx == seg_row).astype(jnp.bfloat16)


def _accumulate(oh, lq_all, bc_acc, i):
    dn = (((1,), (1,)), ((), ()))
    bc = jax.lax.dot_general(lq_all, oh, dn, preferred_element_type=jnp.float32)

    @pl.when(i == 0)
    def _init():
        bc_acc[...] = bc

    @pl.when(i != 0)
    def _acc():
        bc_acc[...] += bc


def _update(sb, bmax, smax, oh, bc, lwt, hw3, ws, wb, comp):
    bf = jnp.bfloat16
    dn = (((1,), (0,)), ((), ()))
    prod_all = jax.lax.dot_general(
        bc.astype(bf), oh, dn,
        preferred_element_type=jnp.float32).astype(bf)
    prod_sp = prod_all[0:C]
    prod_io = prod_all[CP:CP + C]
    prod_att = bf(L - 1.0) * prod_sp

    a_sub = _a_sub(sb, bmax, smax)
    q_mod = sb + (sb == 0).astype(bf)
    log_qmod = jnp.log(q_mod + bf(1e-5))
    log_qsum = jnp.log(a_sub + bf(1e-5))

    ft_sp = jnp.exp(prod_sp - log_qmod)
    ft_io = jnp.exp(prod_io - log_qsum)
    ft_att = jnp.exp(prod_att - log_qmod)

    hw0 = hw3[0:1, 0:1]
    hw1 = hw3[0:1, 1:2]
    hw2 = hw3[0:1, 2:3]
    d0 = (lwt[:, 0:1] - hw0).astype(bf)
    d1 = (lwt[:, 1:2] - hw1).astype(bf)
    d2 = (lwt[:, 2:3] - hw2).astype(bf)
    ftsum = (d0 * ft_sp + d1 * ft_io) + d2 * ft_att
    upd = (hw0 + hw1 + hw2) + ftsum.astype(jnp.float32)

    mw = jnp.dot(comp, ws + wb, preferred_element_type=jnp.float32)
    pw = jnp.dot(mw.astype(bf), sb, preferred_element_type=jnp.float32)
    return pw + upd


def _kernel_all(u_ref, seg_ref, lwt_ref, hw_ref, ws_ref, wb_ref, comp_ref,
                out_ref, u_all, s1_all, smax1_all, s2_all, smax2_all, bc1_acc,
                bc2_acc):
    ph = pl.program_id(0)
    i = pl.program_id(1)
    sl = pl.ds(i * BN, BN)
    oh = _onehot(seg_ref[0])

    @pl.when(ph == 0)
    def _p0():
        u = u_ref[...]
        u_all[:, sl] = u
        s, smax = _softmax0(u)
        sb = s.astype(jnp.bfloat16)
        smax_b = smax.astype(jnp.bfloat16)
        s1_all[:, sl] = sb
        smax1_all[:, sl] = smax_b
        lq_all = _log_terms(sb, sb == smax_b, smax_b)
        _accumulate(oh, lq_all, bc1_acc, i)

    @pl.when(ph == 1)
    def _p1():
        sb = s1_all[:, sl]
        smax_b = smax1_all[:, sl]
        delta = _update(sb, sb == smax_b, smax_b, oh, bc1_acc[...],
                        lwt_ref[...], hw_ref[...], ws_ref[...], wb_ref[...],
                        comp_ref[...])
        q1 = u_all[:, sl] - delta
        s2, smax2 = _softmax0(q1)
        sb2 = s2.astype(jnp.bfloat16)
        smax2_b = smax2.astype(jnp.bfloat16)
        s2_all[:, sl] = sb2
        smax2_all[:, sl] = smax2_b
        lq_all2 = _log_terms(sb2, sb2 == smax2_b, smax2_b)
        _accumulate(oh, lq_all2, bc2_acc, i)

    @pl.when(ph == 2)
    def _p2():
        sb = s2_all[:, sl]
        smax_b = smax2_all[:, sl]
        delta = _update(sb, sb == smax_b, smax_b, oh, bc2_acc[...],
                        lwt_ref[...], hw_ref[...], ws_ref[...], wb_ref[...],
                        comp_ref[...])
        out_ref[...] = u_all[:, sl] - delta


def _crf_single(u_t, seg_row, lwt, hw3, ws, wb, comp, n, interpret=False):
    nb = n // BN
    uspec = pl.BlockSpec((C, BN), lambda ph, i: (0, i * (ph == 0)))
    outspec = pl.BlockSpec((C, BN), lambda ph, i: (0, i * (ph == 2)))
    segspec = pl.BlockSpec((1, 1, BN), lambda ph, i: (i, 0, 0))
    full = lambda a: pl.BlockSpec(a.shape, lambda ph, i: (0,) * a.ndim)

    return pl.pallas_call(
        _kernel_all,
        grid=(3, nb),
        in_specs=[uspec, segspec, full(lwt), full(hw3), full(ws), full(wb),
                  full(comp)],
        out_specs=outspec,
        out_shape=jax.ShapeDtypeStruct((C, n), jnp.float32),
        scratch_shapes=[
            pltpu.VMEM((C, n), jnp.float32),
            pltpu.VMEM((C, n), jnp.bfloat16),
            pltpu.VMEM((1, n), jnp.bfloat16),
            pltpu.VMEM((C, n), jnp.bfloat16),
            pltpu.VMEM((1, n), jnp.bfloat16),
            pltpu.VMEM((2 * CP, LP), jnp.float32),
            pltpu.VMEM((2 * CP, LP), jnp.float32),
        ],
        interpret=interpret,
    )(u_t, seg_row, lwt, hw3, ws, wb, comp)


@functools.partial(jax.jit, static_argnames=("interpret",))
def _run(unary, sp_map, spatial_ker_weights, bilateral_ker_weights,
         complex_rel_low_weights, complex_rel_high_weights,
         compatibility_matrix, interpret=False):
    b, h, w, c = unary.shape
    n = h * w
    lwt = complex_rel_low_weights.T
    hw3 = complex_rel_high_weights.reshape(1, 3)
    outs = []
    for j in range(b):
        u_t = jnp.transpose(unary[j], (2, 0, 1)).reshape(c, n)
        seg_row = sp_map[j].reshape(n // BN, 1, BN).astype(jnp.int32)
        o = _crf_single(u_t, seg_row, lwt, hw3, spatial_ker_weights,
                        bilateral_ker_weights, compatibility_matrix, n,
                        interpret=interpret)
        outs.append(o.reshape(c, h, w))
    return jnp.stack(outs)


def kernel(unary, rgb, sp_map, spatial_ker_weights, bilateral_ker_weights,
           complex_rel_low_weights, complex_rel_high_weights,
           compatibility_matrix):
    return _run(unary, sp_map, spatial_ker_weights, bilateral_ker_weights,
                complex_rel_low_weights, complex_rel_high_weights,
                compatibility_matrix)
